# baseline (speedup 1.0000x reference)
import functools

import jax
import jax.numpy as jnp
from jax import lax
from jax.experimental import pallas as pl
from jax.experimental.pallas import tpu as pltpu
from jax.experimental.pallas import tpu_sc as plsc

B, S, D, E, L, V = 128, 512, 768, 6, 2, 30000

_N_TC = 32
_N_SC = B - _N_TC
_RPW = 8
_NW_SC = _N_SC // _RPW


@functools.cache
def _make_sc_gather():
    nc = 2
    mesh = plsc.VectorSubcoreMesh(
        core_axis_name="c", subcore_axis_name="s", num_cores=nc, num_subcores=16
    )

    @functools.partial(
        pl.kernel,
        mesh=mesh,
        out_type=jax.ShapeDtypeStruct((_N_SC, D), jnp.float32),
        scratch_types=[
            pltpu.VMEM((_RPW,), jnp.int32),
            pltpu.VMEM((_RPW, D), jnp.float32),
            pltpu.SemaphoreType.DMA,
        ],
    )
    def sc_gather(idx_hbm, table_hbm, out_hbm, idx_v, rows_v, sem):
        wid = lax.axis_index("s") * nc + lax.axis_index("c")

        @pl.when(wid < _NW_SC)
        def _():
            base = wid * _RPW
            pltpu.sync_copy(idx_hbm.at[pl.ds(base, _RPW)], idx_v)
            pltpu.async_copy(table_hbm.at[idx_v], rows_v, sem).wait()
            pltpu.sync_copy(rows_v, out_hbm.at[pl.ds(base, _RPW)])

    return sc_gather


_GRP = 8


def _tc_gather_body(idx_s, *refs):
    blocks = refs[:_GRP]
    out_ref = refs[_GRP]
    g = pl.program_id(0)
    rows = []
    for k in range(_GRP):
        rmod = idx_s[g * _GRP + k] % _GRP
        sel = (lax.broadcasted_iota(jnp.int32, (1, _GRP), 1) == rmod).astype(
            jnp.float32
        )
        rows.append(
            jnp.dot(sel, blocks[k][...], preferred_element_type=jnp.float32)
        )
    out_ref[...] = jnp.concatenate(rows, axis=0)


def _tc_gather(idx_tc, embed_table):
    def tbl_spec(k):
        return pl.BlockSpec(
            (_GRP, D), lambda g, idx_ref, k=k: (idx_ref[g * _GRP + k] // _GRP, 0)
        )

    return pl.pallas_call(
        _tc_gather_body,
        grid_spec=pltpu.PrefetchScalarGridSpec(
            num_scalar_prefetch=1,
            grid=(_N_TC // _GRP,),
            in_specs=[tbl_spec(k) for k in range(_GRP)],
            out_specs=pl.BlockSpec((_GRP, D), lambda g, idx_ref: (g, 0)),
        ),
        out_shape=jax.ShapeDtypeStruct((_N_TC, D), jnp.float32),
    )(idx_tc, *([embed_table] * _GRP))


def _moe_head(top_ref, bot_ref, mask_ref, gw_ref, gb_ref, ew_ref, eb_ref, out_ref):
    cls = jnp.concatenate([top_ref[...], bot_ref[...]], axis=0) * mask_ref[...]
    gl = jnp.dot(cls, gw_ref[...], preferred_element_type=jnp.float32) + gb_ref[...]
    eo = jnp.dot(cls, ew_ref[...], preferred_element_type=jnp.float32) + eb_ref[...]
    mx = jnp.max(gl, axis=1, keepdims=True)
    iota_e = lax.broadcasted_iota(jnp.int32, (B, E), 1)
    choice = jnp.min(jnp.where(gl >= mx, iota_e, E), axis=1, keepdims=True)
    iota_el = lax.broadcasted_iota(jnp.int32, (B, E * L), 1)
    o0 = jnp.sum(jnp.where(iota_el == L * choice, eo, 0.0), axis=1, keepdims=True)
    o1 = jnp.sum(jnp.where(iota_el == L * choice + 1, eo, 0.0), axis=1, keepdims=True)
    iota_l = lax.broadcasted_iota(jnp.int32, (B, L), 1)
    out_ref[...] = jnp.where(iota_l == 0, o0, o1)


def kernel(input_ids, attention_mask, embed_table, gate_W, gate_b, experts_W, experts_b):
    mask_col = attention_mask[:, 0:1].astype(jnp.float32)
    ew2 = jnp.transpose(experts_W, (1, 0, 2)).reshape(D, E * L)
    gb2 = gate_b.reshape(1, E)
    eb2 = experts_b.reshape(1, E * L)
    idx = input_ids[:, 0]
    idx_tc = idx[:_N_TC]
    idx_sc = idx[_N_TC:]

    cls_bot = _make_sc_gather()(idx_sc, embed_table)
    cls_top = _tc_gather(idx_tc, embed_table)

    return pl.pallas_call(
        _moe_head,
        out_shape=jax.ShapeDtypeStruct((B, L), jnp.float32),
    )(cls_top, cls_bot, mask_col, gate_W, gb2, ew2, eb2)
